# baseline (speedup 1.0000x reference)
import jax
import jax.numpy as jnp
from jax.experimental import pallas as pl
from jax.experimental.pallas import tpu as pltpu

_N = 10000
_BR = 1024
_NB = 10
_NP = _BR * _NB


def _mega_body(w0_ref, b0_ref, w1_ref, b1_ref, q_ref, dis_ref, y_ref,
               u_ref, r_ref, h1_ref, y1_ref):
    p = pl.program_id(0)
    i = pl.program_id(1)

    @pl.when(jnp.logical_and(p == 0, i == 0))
    def _init():
        u_ref[0, :] = dis_ref[0, :_N].astype(jnp.bfloat16)

    q = q_ref[0, :, :].astype(jnp.float32)
    u = u_ref[0, :].astype(jnp.float32)
    t = jnp.sum(q * u[None, :], axis=1)
    r_ref[0, pl.ds(i * _BR, _BR)] = t

    @pl.when(i == _NB - 1)
    def _boundary():
        dis = dis_ref[0, :]
        r = r_ref[0, :]

        @pl.when(p == 0)
        def _p0():
            h1 = -(dis * r)
            h1_ref[0, :] = h1
            u_ref[0, :] = (dis * h1)[:_N].astype(jnp.bfloat16)

        @pl.when(p == 1)
        def _p1():
            h1 = h1_ref[0, :]
            lth1 = -(dis * r)
            h2 = 2.0 * lth1 - 1.0
            y1 = ((1.0 * w0_ref[0] + h1 * w0_ref[1]) + h2 * w0_ref[2]) \
                + b0_ref[0]
            y1_ref[0, :] = y1
            u_ref[0, :] = (dis * y1)[:_N].astype(jnp.bfloat16)

        @pl.when(p == 2)
        def _p2():
            g1 = -(dis * r)
            h1_ref[0, :] = g1
            u_ref[0, :] = (dis * g1)[:_N].astype(jnp.bfloat16)

        @pl.when(p == 3)
        def _p3():
            y1 = y1_ref[0, :]
            g1 = h1_ref[0, :]
            ltg1 = -(dis * r)
            g2 = 2.0 * ltg1 - y1
            y2 = ((y1 * w1_ref[0] + g1 * w1_ref[1]) + g2 * w1_ref[2]) \
                + b1_ref[0]
            y_ref[0, :] = (jnp.maximum(y2, 0.0) + 0.001)[:_N]


def _impl(xin, W0, b0, W1, b1, interpret=False):
    n = _N

    q4 = xin.astype(jnp.bfloat16)
    q2 = jnp.reshape(q4, (n, n))
    qp = jnp.pad(q2, ((0, _NP - n), (0, 0)))
    q = jnp.reshape(qp, (_NB, _BR, n))

    deg = jnp.sum(xin, axis=(0, 2, 3))
    dis = jnp.where(deg > 0, 1.0 / jnp.sqrt(jnp.maximum(deg, 1e-12)), 0.0)
    disp = jnp.pad(dis, (0, _NP - n))

    y = pl.pallas_call(
        _mega_body,
        grid=(4, _NB),
        in_specs=[
            pl.BlockSpec(memory_space=pltpu.SMEM),
            pl.BlockSpec(memory_space=pltpu.SMEM),
            pl.BlockSpec(memory_space=pltpu.SMEM),
            pl.BlockSpec(memory_space=pltpu.SMEM),
            pl.BlockSpec((1, _BR, n), lambda p, i: (i, 0, 0)),
            pl.BlockSpec((1, _NP), lambda p, i: (0, 0)),
        ],
        out_specs=pl.BlockSpec((1, n), lambda p, i: (0, 0)),
        out_shape=jax.ShapeDtypeStruct((1, n), jnp.float32),
        scratch_shapes=[
            pltpu.VMEM((1, n), jnp.bfloat16),
            pltpu.VMEM((1, _NP), jnp.float32),
            pltpu.VMEM((1, _NP), jnp.float32),
            pltpu.VMEM((1, _NP), jnp.float32),
        ],
        interpret=interpret,
    )(jnp.reshape(W0, (3,)), b0, jnp.reshape(W1, (3,)), b1, q,
      jnp.reshape(disp, (1, _NP)))

    return y


def kernel(xin, W0, b0, W1, b1):
    return _impl(xin, W0, b0, W1, b1)
